# Initial kernel scaffold; baseline (speedup 1.0000x reference)
import jax
import jax.numpy as jnp
from jax.experimental import pallas as pl


def kernel(x, emb_table):
    raise NotImplementedError("write your pallas kernel here")


# speedup vs baseline: 1.4634x; 1.4634x over previous
import jax
import jax.numpy as jnp
from jax.experimental import pallas as pl


def _add_kernel(x_ref, emb_ref, o_ref):
    o_ref[...] = x_ref[...] + emb_ref[...]


def kernel(x, emb_table):
    batch, seq_len, d_model = x.shape
    sb = 256
    grid = (seq_len // sb, batch)
    return pl.pallas_call(
        _add_kernel,
        grid=grid,
        in_specs=[
            pl.BlockSpec((1, sb, d_model), lambda s, b: (b, s, 0)),
            pl.BlockSpec((sb, d_model), lambda s, b: (s, 0)),
        ],
        out_specs=pl.BlockSpec((1, sb, d_model), lambda s, b: (b, s, 0)),
        out_shape=jax.ShapeDtypeStruct(x.shape, x.dtype),
    )(x, emb_table)


# speedup vs baseline: 1.9322x; 1.3204x over previous
import jax
import jax.numpy as jnp
from jax.experimental import pallas as pl


def _add_kernel(x_ref, emb_ref, o_ref):
    o_ref[...] = x_ref[...] + emb_ref[...]


def kernel(x, emb_table):
    batch, seq_len, d_model = x.shape
    sb = 512
    grid = (seq_len // sb, batch)
    return pl.pallas_call(
        _add_kernel,
        grid=grid,
        in_specs=[
            pl.BlockSpec((1, sb, d_model), lambda s, b: (b, s, 0)),
            pl.BlockSpec((sb, d_model), lambda s, b: (s, 0)),
        ],
        out_specs=pl.BlockSpec((1, sb, d_model), lambda s, b: (b, s, 0)),
        out_shape=jax.ShapeDtypeStruct(x.shape, x.dtype),
    )(x, emb_table)


# speedup vs baseline: 2.1138x; 1.0940x over previous
import jax
import jax.numpy as jnp
from jax.experimental import pallas as pl


def _add_kernel(x_ref, emb_ref, o_ref):
    o_ref[...] = x_ref[...] + emb_ref[...]


def kernel(x, emb_table):
    batch, seq_len, d_model = x.shape
    sb = 1024
    grid = (seq_len // sb, batch)
    return pl.pallas_call(
        _add_kernel,
        grid=grid,
        in_specs=[
            pl.BlockSpec((1, sb, d_model), lambda s, b: (b, s, 0)),
            pl.BlockSpec((sb, d_model), lambda s, b: (s, 0)),
        ],
        out_specs=pl.BlockSpec((1, sb, d_model), lambda s, b: (b, s, 0)),
        out_shape=jax.ShapeDtypeStruct(x.shape, x.dtype),
    )(x, emb_table)


# speedup vs baseline: 2.2828x; 1.0800x over previous
import jax
import jax.numpy as jnp
from jax.experimental import pallas as pl


def _add_kernel(x_ref, emb_ref, o_ref):
    o_ref[...] = x_ref[...] + emb_ref[...]


def kernel(x, emb_table):
    batch, seq_len, d_model = x.shape
    sb = 2048
    grid = (seq_len // sb, batch)
    return pl.pallas_call(
        _add_kernel,
        grid=grid,
        in_specs=[
            pl.BlockSpec((1, sb, d_model), lambda s, b: (b, s, 0)),
            pl.BlockSpec((sb, d_model), lambda s, b: (s, 0)),
        ],
        out_specs=pl.BlockSpec((1, sb, d_model), lambda s, b: (b, s, 0)),
        out_shape=jax.ShapeDtypeStruct(x.shape, x.dtype),
    )(x, emb_table)
